# baseline (speedup 1.0000x reference)
import functools

import jax
import jax.numpy as jnp
from jax import lax
from jax.experimental import pallas as pl
from jax.experimental.pallas import tpu as pltpu
from jax.experimental.pallas import tpu_sc as plsc

DELTA_VAR = 0.5
DELTA_DIST = 1.5
ALPHA = 1.0
BETA = 1.0
GAMMA = 0.001
KSEG = 16
EPS = 1e-12

C = 2048
PPW = 32768
NCH = PPW // C
NE = 32


def _sc_body(emb_hbm, mask_hbm, out_hbm,
             tile, mvec, accb, acc, varb, red, tmp, ctab, csqv, varv, tmp16,
             shared, sem):
    cidx = lax.axis_index("c")
    s = lax.axis_index("s")
    grp = s // 8
    b = 2 * cidx + grp
    slot = s % 8
    base_px = slot * PPW

    zeros16 = jnp.zeros((16,), jnp.float32)
    ones16 = jnp.ones((16,), jnp.float32)
    laneoff = lax.iota(jnp.int32, 16) * 528
    vlaneoff = lax.iota(jnp.int32, 16) * 16

    def load_chunk(i):
        base = base_px + i * C
        pltpu.sync_copy(mask_hbm.at[b, pl.ds(base, C)], mvec)
        hs = []
        for e in range(NE):
            hs.append(
                pltpu.async_copy(emb_hbm.at[b, e, pl.ds(base, C)], tile.at[e], sem))
            if e >= 8:
                hs[e - 8].wait()
        for h in hs[NE - 8:]:
            h.wait()

    def zb(v, carry):
        accb[pl.ds(v * 16, 16)] = zeros16
        return carry

    lax.fori_loop(0, 33 * 16, zb, 0)

    def p1_j(j, carry):
        mm = laneoff + mvec[pl.ds(j * 16, 16)]
        for e in range(NE):
            x = tile[e, pl.ds(j * 16, 16)]
            plsc.addupdate_scatter(accb, [mm + (e * 16)], x)
        plsc.addupdate_scatter(accb, [mm + (NE * 16)], ones16)
        return carry

    def p1_chunk(i, carry):
        load_chunk(i)
        lax.fori_loop(0, C // 16, p1_j, 0)
        return carry

    lax.fori_loop(0, NCH, p1_chunk, 0)

    for v in range(33):
        sacc = zeros16
        for l in range(16):
            sacc = sacc + accb[pl.ds(l * 528 + v * 16, 16)]
        acc[pl.ds(v * 16, 16)] = sacc

    pltpu.sync_copy(acc, shared.at[pl.ds(s * 640, 560)])
    plsc.subcore_barrier()

    lo = grp * 8
    for v in range(33):
        red[pl.ds(v * 16, 16)] = zeros16
    for i in range(8):
        pltpu.sync_copy(shared.at[pl.ds((lo + i) * 640, 560)], tmp)
        for v in range(33):
            red[pl.ds(v * 16, 16)] = red[pl.ds(v * 16, 16)] + tmp[pl.ds(v * 16, 16)]

    counts = red[pl.ds(32 * 16, 16)]
    safe = jnp.where(counts > 0.0, counts, 1.0)
    inv = 1.0 / safe
    csq = zeros16
    for e in range(NE):
        ce = red[pl.ds(e * 16, 16)] * inv
        ctab[pl.ds(e * 16, 16)] = ce
        csq = csq + ce * ce
    csqv[...] = csq

    for l in range(16):
        varb[pl.ds(l * 16, 16)] = zeros16

    def p2_j(j, carry):
        mm = mvec[pl.ds(j * 16, 16)]
        normsq = zeros16
        dot = zeros16
        for e in range(NE):
            x = tile[e, pl.ds(j * 16, 16)]
            cg = plsc.load_gather(ctab, [mm + (e * 16)])
            normsq = normsq + x * x
            dot = dot + x * cg
        csqm = plsc.load_gather(csqv, [mm])
        sq = jnp.maximum(normsq - 2.0 * dot + csqm, 0.0) + EPS
        ibits = plsc.bitcast(sq, jnp.int32)
        ibits = jnp.int32(0x5F3759DF) - lax.shift_right_logical(ibits, 1)
        r = plsc.bitcast(ibits, jnp.float32)
        r = r * (1.5 - 0.5 * sq * r * r)
        r = r * (1.5 - 0.5 * sq * r * r)
        r = r * (1.5 - 0.5 * sq * r * r)
        r = r * (1.5 - 0.5 * sq * r * r)
        d = sq * r
        h = jnp.maximum(d - DELTA_VAR, 0.0)
        plsc.addupdate_scatter(varb, [vlaneoff + mm], h * h)
        return carry

    def p2_chunk(i, carry):
        load_chunk(i)
        lax.fori_loop(0, C // 16, p2_j, 0)
        return carry

    lax.fori_loop(0, NCH, p2_chunk, 0)

    varred16 = zeros16
    for l in range(16):
        varred16 = varred16 + varb[pl.ds(l * 16, 16)]
    varv[...] = varred16
    pltpu.sync_copy(varv, shared.at[pl.ds(s * 640 + 528, 16)])
    plsc.subcore_barrier()

    @pl.when(slot == 0)
    def _():
        varred = zeros16
        for i in range(8):
            pltpu.sync_copy(shared.at[pl.ds((lo + i) * 640 + 528, 16)], tmp16)
            varred = varred + tmp16[...]
        red[pl.ds(33 * 16, 16)] = varred
        red[pl.ds(34 * 16, 16)] = zeros16
        pltpu.sync_copy(red, out_hbm.at[b])


def _finish_body(stat_ref, out_ref):
    kk_row = lax.broadcasted_iota(jnp.int32, (1, KSEG), 1)
    kk_sq_r = lax.broadcasted_iota(jnp.int32, (KSEG, KSEG), 1)
    kk_sq_c = lax.broadcasted_iota(jnp.int32, (KSEG, KSEG), 0)
    eye = (kk_sq_c == kk_sq_r).astype(jnp.float32)
    lv_acc = jnp.float32(0.0)
    ld_acc = jnp.float32(0.0)
    lr_acc = jnp.float32(0.0)
    vb_acc = jnp.float32(0.0)
    for bi in range(4):
        stat = stat_ref[bi]
        counts = stat[32:33, :]
        varsum = stat[33:34, :]
        valid_row = jnp.logical_and(counts > 0, kk_row > 0)
        vrf = valid_row.astype(jnp.float32)
        n_inst = jnp.sum(vrf)
        safe = jnp.where(counts > 0, counts, 1.0)
        centers = stat[0:32, :] / safe
        var_per = varsum / safe
        lv = jnp.sum(jnp.where(valid_row, var_per, 0.0)) / jnp.maximum(n_inst, 1.0)
        csq_row = jnp.sum(centers * centers, axis=0, keepdims=True)
        gram = lax.dot_general(centers, centers, (((0,), (0,)), ((), ())),
                               preferred_element_type=jnp.float32)
        csq_col = jnp.sum(eye * gram, axis=1, keepdims=True)
        sq_pair = jnp.maximum(csq_col + csq_row - 2.0 * gram, 0.0)
        outer = lax.dot_general(vrf, vrf, (((0,), (0,)), ((), ())),
                                preferred_element_type=jnp.float32)
        pm = jnp.logical_and(outer > 0.5, kk_sq_c < kk_sq_r)
        pair_d = jnp.sqrt(jnp.where(pm, sq_pair, 1.0))
        hd = jnp.maximum(2.0 * DELTA_DIST - pair_d, 0.0) ** 2
        n_pairs = jnp.sum(pm.astype(jnp.float32))
        ld = jnp.sum(jnp.where(pm, hd, 0.0)) / jnp.maximum(n_pairs, 1.0)
        c_norm = jnp.sqrt(jnp.where(valid_row, csq_row, 1.0))
        lr = jnp.sum(jnp.where(valid_row, c_norm, 0.0)) / jnp.maximum(n_inst, 1.0)
        validb = (n_inst > 0).astype(jnp.float32)
        lv_acc += lv * validb
        ld_acc += ld * validb
        lr_acc += lr * validb
        vb_acc += validb
    denom = jnp.maximum(vb_acc, 1.0)
    lvt = lv_acc / denom
    ldt = ld_acc / denom
    lrt = lr_acc / denom
    total = ALPHA * lvt + BETA * ldt + GAMMA * lrt
    row = lax.broadcasted_iota(jnp.int32, (8, 128), 0)
    col = lax.broadcasted_iota(jnp.int32, (8, 128), 1)
    vals = jnp.where(col == 0, total,
           jnp.where(col == 1, lvt,
           jnp.where(col == 2, ldt, lrt)))
    out_ref[...] = jnp.where(row == 0, vals, 0.0)


def kernel(embedding, instance_mask):
    if instance_mask.ndim == 4:
        instance_mask = instance_mask[:, 0]
    B, E, H, W = embedding.shape
    P = H * W
    emb3 = embedding.reshape(B, E, P)
    mask2 = instance_mask.reshape(B, P)

    mesh = plsc.VectorSubcoreMesh(core_axis_name="c", subcore_axis_name="s")
    sc_call = functools.partial(
        pl.kernel,
        mesh=mesh,
        compiler_params=pltpu.CompilerParams(needs_layout_passes=False),
        out_type=jax.ShapeDtypeStruct((B, 560), jnp.float32),
        scratch_types=[
            pltpu.VMEM((NE, C), jnp.float32),
            pltpu.VMEM((C,), jnp.int32),
            pltpu.VMEM((16 * 528,), jnp.float32),
            pltpu.VMEM((560,), jnp.float32),
            pltpu.VMEM((16 * 16,), jnp.float32),
            pltpu.VMEM((560,), jnp.float32),
            pltpu.VMEM((560,), jnp.float32),
            pltpu.VMEM((NE * 16,), jnp.float32),
            pltpu.VMEM((16,), jnp.float32),
            pltpu.VMEM((16,), jnp.float32),
            pltpu.VMEM((16,), jnp.float32),
            pltpu.VMEM_SHARED((16 * 640,), jnp.float32),
            pltpu.SemaphoreType.DMA,
        ],
    )(_sc_body)
    stats = sc_call(emb3, mask2).reshape(B, 35, 16)

    out = pl.pallas_call(
        _finish_body,
        grid=(1,),
        in_specs=[pl.BlockSpec((B, 35, 16), lambda i: (0, 0, 0))],
        out_specs=pl.BlockSpec((8, 128), lambda i: (0, 0)),
        out_shape=jax.ShapeDtypeStruct((8, 128), jnp.float32),
    )(stats)
    return (out[0, 0], out[0, 1], out[0, 2], out[0, 3])


# speedup vs baseline: 1.0044x; 1.0044x over previous
import functools

import jax
import jax.numpy as jnp
from jax import lax
from jax.experimental import pallas as pl
from jax.experimental.pallas import tpu as pltpu
from jax.experimental.pallas import tpu_sc as plsc

DELTA_VAR = 0.5
DELTA_DIST = 1.5
ALPHA = 1.0
BETA = 1.0
GAMMA = 0.001
KSEG = 16
EPS = 1e-12

C = 2048
PPW = 32768
NCH = PPW // C
NE = 32


def _sc_body(emb_hbm, mask_hbm, out_hbm,
             tile, mvec, accb, acc, varb, red, tmp, ctab, csqv, varv, tmp16,
             shared, sem):
    cidx = lax.axis_index("c")
    s = lax.axis_index("s")
    grp = s // 8
    b = 2 * cidx + grp
    slot = s % 8
    base_px = slot * PPW

    zeros16 = jnp.zeros((16,), jnp.float32)
    ones16 = jnp.ones((16,), jnp.float32)
    laneoff = lax.iota(jnp.int32, 16) * 528
    vlaneoff = lax.iota(jnp.int32, 16) * 16

    def load_chunk(i):
        base = base_px + i * C
        pltpu.sync_copy(mask_hbm.at[b, pl.ds(base, C)], mvec)
        hs = []
        for e in range(NE):
            hs.append(
                pltpu.async_copy(emb_hbm.at[b, e, pl.ds(base, C)], tile.at[e], sem))
            if e >= 8:
                hs[e - 8].wait()
        for h in hs[NE - 8:]:
            h.wait()

    def zb(v, carry):
        accb[pl.ds(v * 16, 16)] = zeros16
        return carry

    lax.fori_loop(0, 33 * 16, zb, 0)

    def p1_j(j, carry):
        mm = laneoff + mvec[pl.ds(j * 16, 16)]
        for e in range(NE):
            x = tile[e, pl.ds(j * 16, 16)]
            plsc.addupdate_scatter(accb, [mm + (e * 16)], x)
        plsc.addupdate_scatter(accb, [mm + (NE * 16)], ones16)
        return carry

    def p1_chunk(i, carry):
        load_chunk(i)
        lax.fori_loop(0, C // 16, p1_j, 0)
        return carry

    lax.fori_loop(0, NCH, p1_chunk, 0)

    for v in range(33):
        sacc = zeros16
        for l in range(16):
            sacc = sacc + accb[pl.ds(l * 528 + v * 16, 16)]
        acc[pl.ds(v * 16, 16)] = sacc

    pltpu.sync_copy(acc, shared.at[pl.ds(s * 640, 560)])
    plsc.subcore_barrier()

    lo = grp * 8
    for v in range(33):
        red[pl.ds(v * 16, 16)] = zeros16
    for i in range(8):
        pltpu.sync_copy(shared.at[pl.ds((lo + i) * 640, 560)], tmp)
        for v in range(33):
            red[pl.ds(v * 16, 16)] = red[pl.ds(v * 16, 16)] + tmp[pl.ds(v * 16, 16)]

    counts = red[pl.ds(32 * 16, 16)]
    safe = jnp.where(counts > 0.0, counts, 1.0)
    inv = 1.0 / safe
    csq = zeros16
    for e in range(NE):
        ce = red[pl.ds(e * 16, 16)] * inv
        ctab[pl.ds(e * 16, 16)] = ce
        csq = csq + ce * ce
    csqv[...] = csq

    for l in range(16):
        varb[pl.ds(l * 16, 16)] = zeros16

    def p2_j(j, carry):
        mm = mvec[pl.ds(j * 16, 16)]
        nacc = [zeros16, zeros16, zeros16, zeros16]
        dacc = [zeros16, zeros16, zeros16, zeros16]
        for e in range(NE):
            x = tile[e, pl.ds(j * 16, 16)]
            cg = plsc.load_gather(ctab, [mm + (e * 16)])
            a = e & 3
            nacc[a] = nacc[a] + x * x
            dacc[a] = dacc[a] + x * cg
        normsq = (nacc[0] + nacc[1]) + (nacc[2] + nacc[3])
        dot = (dacc[0] + dacc[1]) + (dacc[2] + dacc[3])
        csqm = plsc.load_gather(csqv, [mm])
        sq = jnp.maximum(normsq - 2.0 * dot + csqm, 0.0) + EPS
        ibits = plsc.bitcast(sq, jnp.int32)
        ibits = jnp.int32(0x5F3759DF) - lax.shift_right_logical(ibits, 1)
        r = plsc.bitcast(ibits, jnp.float32)
        r = r * (1.5 - 0.5 * sq * r * r)
        r = r * (1.5 - 0.5 * sq * r * r)
        r = r * (1.5 - 0.5 * sq * r * r)
        d = sq * r
        h = jnp.maximum(d - DELTA_VAR, 0.0)
        plsc.addupdate_scatter(varb, [vlaneoff + mm], h * h)
        return carry

    def p2_chunk(i, carry):
        load_chunk(i)
        lax.fori_loop(0, C // 16, p2_j, 0)
        return carry

    lax.fori_loop(0, NCH, p2_chunk, 0)

    varred16 = zeros16
    for l in range(16):
        varred16 = varred16 + varb[pl.ds(l * 16, 16)]
    varv[...] = varred16
    pltpu.sync_copy(varv, shared.at[pl.ds(s * 640 + 528, 16)])
    plsc.subcore_barrier()

    @pl.when(slot == 0)
    def _():
        varred = zeros16
        for i in range(8):
            pltpu.sync_copy(shared.at[pl.ds((lo + i) * 640 + 528, 16)], tmp16)
            varred = varred + tmp16[...]
        red[pl.ds(33 * 16, 16)] = varred
        red[pl.ds(34 * 16, 16)] = zeros16
        pltpu.sync_copy(red, out_hbm.at[b])


def _finish_body(stat_ref, out_ref):
    kk_row = lax.broadcasted_iota(jnp.int32, (1, KSEG), 1)
    kk_sq_r = lax.broadcasted_iota(jnp.int32, (KSEG, KSEG), 1)
    kk_sq_c = lax.broadcasted_iota(jnp.int32, (KSEG, KSEG), 0)
    eye = (kk_sq_c == kk_sq_r).astype(jnp.float32)
    lv_acc = jnp.float32(0.0)
    ld_acc = jnp.float32(0.0)
    lr_acc = jnp.float32(0.0)
    vb_acc = jnp.float32(0.0)
    for bi in range(4):
        stat = stat_ref[bi]
        counts = stat[32:33, :]
        varsum = stat[33:34, :]
        valid_row = jnp.logical_and(counts > 0, kk_row > 0)
        vrf = valid_row.astype(jnp.float32)
        n_inst = jnp.sum(vrf)
        safe = jnp.where(counts > 0, counts, 1.0)
        centers = stat[0:32, :] / safe
        var_per = varsum / safe
        lv = jnp.sum(jnp.where(valid_row, var_per, 0.0)) / jnp.maximum(n_inst, 1.0)
        csq_row = jnp.sum(centers * centers, axis=0, keepdims=True)
        gram = lax.dot_general(centers, centers, (((0,), (0,)), ((), ())),
                               preferred_element_type=jnp.float32)
        csq_col = jnp.sum(eye * gram, axis=1, keepdims=True)
        sq_pair = jnp.maximum(csq_col + csq_row - 2.0 * gram, 0.0)
        outer = lax.dot_general(vrf, vrf, (((0,), (0,)), ((), ())),
                                preferred_element_type=jnp.float32)
        pm = jnp.logical_and(outer > 0.5, kk_sq_c < kk_sq_r)
        pair_d = jnp.sqrt(jnp.where(pm, sq_pair, 1.0))
        hd = jnp.maximum(2.0 * DELTA_DIST - pair_d, 0.0) ** 2
        n_pairs = jnp.sum(pm.astype(jnp.float32))
        ld = jnp.sum(jnp.where(pm, hd, 0.0)) / jnp.maximum(n_pairs, 1.0)
        c_norm = jnp.sqrt(jnp.where(valid_row, csq_row, 1.0))
        lr = jnp.sum(jnp.where(valid_row, c_norm, 0.0)) / jnp.maximum(n_inst, 1.0)
        validb = (n_inst > 0).astype(jnp.float32)
        lv_acc += lv * validb
        ld_acc += ld * validb
        lr_acc += lr * validb
        vb_acc += validb
    denom = jnp.maximum(vb_acc, 1.0)
    lvt = lv_acc / denom
    ldt = ld_acc / denom
    lrt = lr_acc / denom
    total = ALPHA * lvt + BETA * ldt + GAMMA * lrt
    row = lax.broadcasted_iota(jnp.int32, (8, 128), 0)
    col = lax.broadcasted_iota(jnp.int32, (8, 128), 1)
    vals = jnp.where(col == 0, total,
           jnp.where(col == 1, lvt,
           jnp.where(col == 2, ldt, lrt)))
    out_ref[...] = jnp.where(row == 0, vals, 0.0)


def kernel(embedding, instance_mask):
    if instance_mask.ndim == 4:
        instance_mask = instance_mask[:, 0]
    B, E, H, W = embedding.shape
    P = H * W
    emb3 = embedding.reshape(B, E, P)
    mask2 = instance_mask.reshape(B, P)

    mesh = plsc.VectorSubcoreMesh(core_axis_name="c", subcore_axis_name="s")
    sc_call = functools.partial(
        pl.kernel,
        mesh=mesh,
        compiler_params=pltpu.CompilerParams(needs_layout_passes=False),
        out_type=jax.ShapeDtypeStruct((B, 560), jnp.float32),
        scratch_types=[
            pltpu.VMEM((NE, C), jnp.float32),
            pltpu.VMEM((C,), jnp.int32),
            pltpu.VMEM((16 * 528,), jnp.float32),
            pltpu.VMEM((560,), jnp.float32),
            pltpu.VMEM((16 * 16,), jnp.float32),
            pltpu.VMEM((560,), jnp.float32),
            pltpu.VMEM((560,), jnp.float32),
            pltpu.VMEM((NE * 16,), jnp.float32),
            pltpu.VMEM((16,), jnp.float32),
            pltpu.VMEM((16,), jnp.float32),
            pltpu.VMEM((16,), jnp.float32),
            pltpu.VMEM_SHARED((16 * 640,), jnp.float32),
            pltpu.SemaphoreType.DMA,
        ],
    )(_sc_body)
    stats = sc_call(emb3, mask2).reshape(B, 35, 16)

    out = pl.pallas_call(
        _finish_body,
        grid=(1,),
        in_specs=[pl.BlockSpec((B, 35, 16), lambda i: (0, 0, 0))],
        out_specs=pl.BlockSpec((8, 128), lambda i: (0, 0)),
        out_shape=jax.ShapeDtypeStruct((8, 128), jnp.float32),
    )(stats)
    return (out[0, 0], out[0, 1], out[0, 2], out[0, 3])


# speedup vs baseline: 1.0839x; 1.0791x over previous
import functools

import jax
import jax.numpy as jnp
from jax import lax
from jax.experimental import pallas as pl
from jax.experimental.pallas import tpu as pltpu
from jax.experimental.pallas import tpu_sc as plsc

DELTA_VAR = 0.5
DELTA_DIST = 1.5
ALPHA = 1.0
BETA = 1.0
GAMMA = 0.001
KSEG = 16
EPS = 1e-12

C = 1024
PPW = 32768
NCH = PPW // C
NPAIR = NCH // 2
NE = 32


def _sc_body(emb_hbm, mask_hbm, out_hbm,
             tile0, tile1, mvec0, mvec1, accb, acc, varb, red, tmp, ctab,
             csqv, varv, tmp16, shared, sem0, sem1):
    cidx = lax.axis_index("c")
    s = lax.axis_index("s")
    grp = s // 8
    b = 2 * cidx + grp
    slot = s % 8
    base_px = slot * PPW

    zeros16 = jnp.zeros((16,), jnp.float32)
    ones16 = jnp.ones((16,), jnp.float32)
    laneoff = lax.iota(jnp.int32, 16) * 528
    vlaneoff = lax.iota(jnp.int32, 16) * 16

    def issue(i, tileb, mvecb, semb):
        base = base_px + i * C
        pltpu.async_copy(mask_hbm.at[b, pl.ds(base, C)], mvecb, semb)
        for e in range(NE):
            pltpu.async_copy(emb_hbm.at[b, e, pl.ds(base, C)], tileb.at[e], semb)

    def drain(tileb, mvecb, semb):
        pltpu.make_async_copy(mask_hbm.at[0, pl.ds(0, C)], mvecb, semb).wait()
        for e in range(NE):
            pltpu.make_async_copy(emb_hbm.at[0, 0, pl.ds(0, C)], tileb.at[e],
                                  semb).wait()

    def run_pass(jbody):
        issue(0, tile0, mvec0, sem0)

        def pair(p, carry):
            i0 = 2 * p
            issue(i0 + 1, tile1, mvec1, sem1)
            drain(tile0, mvec0, sem0)
            lax.fori_loop(0, C // 16,
                          lambda j, c2: jbody(tile0, mvec0, j, c2), 0)

            @pl.when(p < NPAIR - 1)
            def _():
                issue(i0 + 2, tile0, mvec0, sem0)

            drain(tile1, mvec1, sem1)
            lax.fori_loop(0, C // 16,
                          lambda j, c2: jbody(tile1, mvec1, j, c2), 0)
            return carry

        lax.fori_loop(0, NPAIR, pair, 0)

    def zb(v, carry):
        accb[pl.ds(v * 16, 16)] = zeros16
        return carry

    lax.fori_loop(0, 33 * 16, zb, 0)

    def p1_j(tileb, mvecb, j, carry):
        mm = laneoff + mvecb[pl.ds(j * 16, 16)]
        for e in range(NE):
            x = tileb[e, pl.ds(j * 16, 16)]
            plsc.addupdate_scatter(accb, [mm + (e * 16)], x)
        plsc.addupdate_scatter(accb, [mm + (NE * 16)], ones16)
        return carry

    run_pass(p1_j)

    for v in range(33):
        sacc = zeros16
        for l in range(16):
            sacc = sacc + accb[pl.ds(l * 528 + v * 16, 16)]
        acc[pl.ds(v * 16, 16)] = sacc

    pltpu.sync_copy(acc, shared.at[pl.ds(s * 640, 560)])
    plsc.subcore_barrier()

    lo = grp * 8
    for v in range(33):
        red[pl.ds(v * 16, 16)] = zeros16
    for i in range(8):
        pltpu.sync_copy(shared.at[pl.ds((lo + i) * 640, 560)], tmp)
        for v in range(33):
            red[pl.ds(v * 16, 16)] = red[pl.ds(v * 16, 16)] + tmp[pl.ds(v * 16, 16)]

    counts = red[pl.ds(32 * 16, 16)]
    safe = jnp.where(counts > 0.0, counts, 1.0)
    inv = 1.0 / safe
    csq = zeros16
    for e in range(NE):
        ce = red[pl.ds(e * 16, 16)] * inv
        ctab[pl.ds(e * 16, 16)] = ce
        csq = csq + ce * ce
    csqv[...] = csq

    for l in range(16):
        varb[pl.ds(l * 16, 16)] = zeros16

    def p2_j(tileb, mvecb, j, carry):
        mm = mvecb[pl.ds(j * 16, 16)]
        nacc = [zeros16, zeros16, zeros16, zeros16]
        dacc = [zeros16, zeros16, zeros16, zeros16]
        for e in range(NE):
            x = tileb[e, pl.ds(j * 16, 16)]
            cg = plsc.load_gather(ctab, [mm + (e * 16)])
            a = e & 3
            nacc[a] = nacc[a] + x * x
            dacc[a] = dacc[a] + x * cg
        normsq = (nacc[0] + nacc[1]) + (nacc[2] + nacc[3])
        dot = (dacc[0] + dacc[1]) + (dacc[2] + dacc[3])
        csqm = plsc.load_gather(csqv, [mm])
        sq = jnp.maximum(normsq - 2.0 * dot + csqm, 0.0) + EPS
        ibits = plsc.bitcast(sq, jnp.int32)
        ibits = jnp.int32(0x5F3759DF) - lax.shift_right_logical(ibits, 1)
        r = plsc.bitcast(ibits, jnp.float32)
        r = r * (1.5 - 0.5 * sq * r * r)
        r = r * (1.5 - 0.5 * sq * r * r)
        r = r * (1.5 - 0.5 * sq * r * r)
        d = sq * r
        h = jnp.maximum(d - DELTA_VAR, 0.0)
        plsc.addupdate_scatter(varb, [vlaneoff + mm], h * h)
        return carry

    run_pass(p2_j)

    varred16 = zeros16
    for l in range(16):
        varred16 = varred16 + varb[pl.ds(l * 16, 16)]
    varv[...] = varred16
    pltpu.sync_copy(varv, shared.at[pl.ds(s * 640 + 528, 16)])
    plsc.subcore_barrier()

    @pl.when(slot == 0)
    def _():
        varred = zeros16
        for i in range(8):
            pltpu.sync_copy(shared.at[pl.ds((lo + i) * 640 + 528, 16)], tmp16)
            varred = varred + tmp16[...]
        red[pl.ds(33 * 16, 16)] = varred
        red[pl.ds(34 * 16, 16)] = zeros16
        pltpu.sync_copy(red, out_hbm.at[b])


def _finish_body(stat_ref, out_ref):
    kk_row = lax.broadcasted_iota(jnp.int32, (1, KSEG), 1)
    kk_sq_r = lax.broadcasted_iota(jnp.int32, (KSEG, KSEG), 1)
    kk_sq_c = lax.broadcasted_iota(jnp.int32, (KSEG, KSEG), 0)
    eye = (kk_sq_c == kk_sq_r).astype(jnp.float32)
    lv_acc = jnp.float32(0.0)
    ld_acc = jnp.float32(0.0)
    lr_acc = jnp.float32(0.0)
    vb_acc = jnp.float32(0.0)
    for bi in range(4):
        stat = stat_ref[bi]
        counts = stat[32:33, :]
        varsum = stat[33:34, :]
        valid_row = jnp.logical_and(counts > 0, kk_row > 0)
        vrf = valid_row.astype(jnp.float32)
        n_inst = jnp.sum(vrf)
        safe = jnp.where(counts > 0, counts, 1.0)
        centers = stat[0:32, :] / safe
        var_per = varsum / safe
        lv = jnp.sum(jnp.where(valid_row, var_per, 0.0)) / jnp.maximum(n_inst, 1.0)
        csq_row = jnp.sum(centers * centers, axis=0, keepdims=True)
        gram = lax.dot_general(centers, centers, (((0,), (0,)), ((), ())),
                               preferred_element_type=jnp.float32)
        csq_col = jnp.sum(eye * gram, axis=1, keepdims=True)
        sq_pair = jnp.maximum(csq_col + csq_row - 2.0 * gram, 0.0)
        outer = lax.dot_general(vrf, vrf, (((0,), (0,)), ((), ())),
                                preferred_element_type=jnp.float32)
        pm = jnp.logical_and(outer > 0.5, kk_sq_c < kk_sq_r)
        pair_d = jnp.sqrt(jnp.where(pm, sq_pair, 1.0))
        hd = jnp.maximum(2.0 * DELTA_DIST - pair_d, 0.0) ** 2
        n_pairs = jnp.sum(pm.astype(jnp.float32))
        ld = jnp.sum(jnp.where(pm, hd, 0.0)) / jnp.maximum(n_pairs, 1.0)
        c_norm = jnp.sqrt(jnp.where(valid_row, csq_row, 1.0))
        lr = jnp.sum(jnp.where(valid_row, c_norm, 0.0)) / jnp.maximum(n_inst, 1.0)
        validb = (n_inst > 0).astype(jnp.float32)
        lv_acc += lv * validb
        ld_acc += ld * validb
        lr_acc += lr * validb
        vb_acc += validb
    denom = jnp.maximum(vb_acc, 1.0)
    lvt = lv_acc / denom
    ldt = ld_acc / denom
    lrt = lr_acc / denom
    total = ALPHA * lvt + BETA * ldt + GAMMA * lrt
    row = lax.broadcasted_iota(jnp.int32, (8, 128), 0)
    col = lax.broadcasted_iota(jnp.int32, (8, 128), 1)
    vals = jnp.where(col == 0, total,
           jnp.where(col == 1, lvt,
           jnp.where(col == 2, ldt, lrt)))
    out_ref[...] = jnp.where(row == 0, vals, 0.0)


def kernel(embedding, instance_mask):
    if instance_mask.ndim == 4:
        instance_mask = instance_mask[:, 0]
    B, E, H, W = embedding.shape
    P = H * W
    emb3 = embedding.reshape(B, E, P)
    mask2 = instance_mask.reshape(B, P)

    mesh = plsc.VectorSubcoreMesh(core_axis_name="c", subcore_axis_name="s")
    sc_call = functools.partial(
        pl.kernel,
        mesh=mesh,
        compiler_params=pltpu.CompilerParams(needs_layout_passes=False),
        out_type=jax.ShapeDtypeStruct((B, 560), jnp.float32),
        scratch_types=[
            pltpu.VMEM((NE, C), jnp.float32),
            pltpu.VMEM((NE, C), jnp.float32),
            pltpu.VMEM((C,), jnp.int32),
            pltpu.VMEM((C,), jnp.int32),
            pltpu.VMEM((16 * 528,), jnp.float32),
            pltpu.VMEM((560,), jnp.float32),
            pltpu.VMEM((16 * 16,), jnp.float32),
            pltpu.VMEM((560,), jnp.float32),
            pltpu.VMEM((560,), jnp.float32),
            pltpu.VMEM((NE * 16,), jnp.float32),
            pltpu.VMEM((16,), jnp.float32),
            pltpu.VMEM((16,), jnp.float32),
            pltpu.VMEM((16,), jnp.float32),
            pltpu.VMEM_SHARED((16 * 640,), jnp.float32),
            pltpu.SemaphoreType.DMA,
            pltpu.SemaphoreType.DMA,
        ],
    )(_sc_body)
    stats = sc_call(emb3, mask2).reshape(B, 35, 16)

    out = pl.pallas_call(
        _finish_body,
        grid=(1,),
        in_specs=[pl.BlockSpec((B, 35, 16), lambda i: (0, 0, 0))],
        out_specs=pl.BlockSpec((8, 128), lambda i: (0, 0)),
        out_shape=jax.ShapeDtypeStruct((8, 128), jnp.float32),
    )(stats)
    return (out[0, 0], out[0, 1], out[0, 2], out[0, 3])


# speedup vs baseline: 2.1265x; 1.9620x over previous
import functools

import jax
import jax.numpy as jnp
from jax import lax
from jax.experimental import pallas as pl
from jax.experimental.pallas import tpu as pltpu
from jax.experimental.pallas import tpu_sc as plsc

DELTA_VAR = 0.5
DELTA_DIST = 1.5
ALPHA = 1.0
BETA = 1.0
GAMMA = 0.001
KSEG = 16
EPS = 1e-12

NE = 32
TILE = 16384
NT_TC = 12
P_TC = NT_TC * TILE
C = 1024
PPW = 8192
NCH = PPW // C
NPAIR = NCH // 2


def _sc_p1_body(emb_hbm, mask_hbm, out_hbm,
                tile0, tile1, mvec0, mvec1, accb, acc, sem0, sem1):
    cidx = lax.axis_index("c")
    s = lax.axis_index("s")
    b = 2 * cidx + s // 8
    wid = s + 16 * cidx
    base_px = P_TC + (s % 8) * PPW

    zeros16 = jnp.zeros((16,), jnp.float32)
    ones16 = jnp.ones((16,), jnp.float32)
    laneoff = lax.iota(jnp.int32, 16) * 528

    def issue(i, tileb, mvecb, semb):
        base = base_px + i * C
        pltpu.async_copy(mask_hbm.at[b, pl.ds(base, C)], mvecb, semb)
        for e in range(NE):
            pltpu.async_copy(emb_hbm.at[b, e, pl.ds(base, C)], tileb.at[e], semb)

    def drain(tileb, mvecb, semb):
        pltpu.make_async_copy(mask_hbm.at[0, pl.ds(0, C)], mvecb, semb).wait()
        for e in range(NE):
            pltpu.make_async_copy(emb_hbm.at[0, 0, pl.ds(0, C)], tileb.at[e],
                                  semb).wait()

    def jbody(tileb, mvecb, j, carry):
        mm = laneoff + mvecb[pl.ds(j * 16, 16)]
        for e in range(NE):
            x = tileb[e, pl.ds(j * 16, 16)]
            plsc.addupdate_scatter(accb, [mm + (e * 16)], x)
        plsc.addupdate_scatter(accb, [mm + (NE * 16)], ones16)
        return carry

    def zb(v, carry):
        accb[pl.ds(v * 16, 16)] = zeros16
        return carry

    lax.fori_loop(0, 33 * 16, zb, 0)

    issue(0, tile0, mvec0, sem0)

    def pair(p, carry):
        i0 = 2 * p
        issue(i0 + 1, tile1, mvec1, sem1)
        drain(tile0, mvec0, sem0)
        lax.fori_loop(0, C // 16, lambda j, c2: jbody(tile0, mvec0, j, c2), 0)

        @pl.when(p < NPAIR - 1)
        def _():
            issue(i0 + 2, tile0, mvec0, sem0)

        drain(tile1, mvec1, sem1)
        lax.fori_loop(0, C // 16, lambda j, c2: jbody(tile1, mvec1, j, c2), 0)
        return carry

    lax.fori_loop(0, NPAIR, pair, 0)

    for v in range(33):
        sacc = zeros16
        for l in range(16):
            sacc = sacc + accb[pl.ds(l * 528 + v * 16, 16)]
        acc[pl.ds(v * 16, 16)] = sacc
    for v in range(33, 35):
        acc[pl.ds(v * 16, 16)] = zeros16
    pltpu.sync_copy(acc, out_hbm.at[wid])


def _sc_p2_body(emb_hbm, mask_hbm, ctab_hbm, out_hbm,
                tile0, tile1, mvec0, mvec1, ctab, varb, varv, sem0, sem1):
    cidx = lax.axis_index("c")
    s = lax.axis_index("s")
    b = 2 * cidx + s // 8
    wid = s + 16 * cidx
    base_px = P_TC + (s % 8) * PPW

    zeros16 = jnp.zeros((16,), jnp.float32)
    vlaneoff = lax.iota(jnp.int32, 16) * 16

    pltpu.sync_copy(ctab_hbm.at[pl.ds(b * 768, 640)], ctab)

    def issue(i, tileb, mvecb, semb):
        base = base_px + i * C
        pltpu.async_copy(mask_hbm.at[b, pl.ds(base, C)], mvecb, semb)
        for e in range(NE):
            pltpu.async_copy(emb_hbm.at[b, e, pl.ds(base, C)], tileb.at[e], semb)

    def drain(tileb, mvecb, semb):
        pltpu.make_async_copy(mask_hbm.at[0, pl.ds(0, C)], mvecb, semb).wait()
        for e in range(NE):
            pltpu.make_async_copy(emb_hbm.at[0, 0, pl.ds(0, C)], tileb.at[e],
                                  semb).wait()

    def jbody(tileb, mvecb, j, carry):
        mm = mvecb[pl.ds(j * 16, 16)]
        nacc = [zeros16, zeros16, zeros16, zeros16]
        dacc = [zeros16, zeros16, zeros16, zeros16]
        for e in range(NE):
            x = tileb[e, pl.ds(j * 16, 16)]
            cg = plsc.load_gather(ctab, [mm + (e * 16)])
            a = e & 3
            nacc[a] = nacc[a] + x * x
            dacc[a] = dacc[a] + x * cg
        normsq = (nacc[0] + nacc[1]) + (nacc[2] + nacc[3])
        dot = (dacc[0] + dacc[1]) + (dacc[2] + dacc[3])
        csqm = plsc.load_gather(ctab, [mm + 512])
        sq = jnp.maximum(normsq - 2.0 * dot + csqm, 0.0) + EPS
        ibits = plsc.bitcast(sq, jnp.int32)
        ibits = jnp.int32(0x5F3759DF) - lax.shift_right_logical(ibits, 1)
        r = plsc.bitcast(ibits, jnp.float32)
        r = r * (1.5 - 0.5 * sq * r * r)
        r = r * (1.5 - 0.5 * sq * r * r)
        r = r * (1.5 - 0.5 * sq * r * r)
        d = sq * r
        h = jnp.maximum(d - DELTA_VAR, 0.0)
        plsc.addupdate_scatter(varb, [vlaneoff + mm], h * h)
        return carry

    for l in range(16):
        varb[pl.ds(l * 16, 16)] = zeros16

    issue(0, tile0, mvec0, sem0)

    def pair(p, carry):
        i0 = 2 * p
        issue(i0 + 1, tile1, mvec1, sem1)
        drain(tile0, mvec0, sem0)
        lax.fori_loop(0, C // 16, lambda j, c2: jbody(tile0, mvec0, j, c2), 0)

        @pl.when(p < NPAIR - 1)
        def _():
            issue(i0 + 2, tile0, mvec0, sem0)

        drain(tile1, mvec1, sem1)
        lax.fori_loop(0, C // 16, lambda j, c2: jbody(tile1, mvec1, j, c2), 0)
        return carry

    lax.fori_loop(0, NPAIR, pair, 0)

    varred = zeros16
    for l in range(16):
        varred = varred + varb[pl.ds(l * 16, 16)]
    varv[...] = varred
    pltpu.sync_copy(varv, out_hbm.at[wid])


def _tc_p1_body(emb_ref, mask_ref, out_ref, sums_s, counts_s):
    t = pl.program_id(1)
    nT = pl.num_programs(1)
    emb = emb_ref[0]
    m = mask_ref[0]
    iota_col = lax.broadcasted_iota(jnp.int32, (KSEG, 1), 0)
    onehot = (m == iota_col).astype(jnp.float32)

    @pl.when(t == 0)
    def _():
        sums_s[...] = jnp.zeros_like(sums_s)
        counts_s[...] = jnp.zeros_like(counts_s)

    sums_s[...] += lax.dot_general(emb, onehot, (((1,), (1,)), ((), ())),
                                   preferred_element_type=jnp.float32)
    counts_s[...] += jnp.sum(onehot, axis=1, keepdims=True).T

    @pl.when(t == nT - 1)
    def _():
        out_ref[0, 0:NE, :] = sums_s[...]
        out_ref[0, NE:NE + 1, :] = counts_s[...]


def _combine_body(tcp1_ref, scp1_ref, out_ref):
    for bi in range(4):
        acc = tcp1_ref[bi, 0:33, :]
        c0 = bi // 2
        s0 = (bi % 2) * 8
        for i in range(8):
            acc = acc + scp1_ref[16 * c0 + s0 + i, 0:33, :]
        counts = acc[32:33, :]
        safe = jnp.where(counts > 0, counts, 1.0)
        centers = acc[0:32, :] / safe
        csq = jnp.sum(centers * centers, axis=0, keepdims=True)
        out_ref[bi, 0:32, :] = centers
        out_ref[bi, 32:33, :] = csq
        out_ref[bi, 40:41, :] = counts


def _tc_p2_body(emb_ref, mask_ref, ctab_ref, out_ref, var_s):
    t = pl.program_id(1)
    nT = pl.num_programs(1)
    emb = emb_ref[0]
    m = mask_ref[0]
    iota_col = lax.broadcasted_iota(jnp.int32, (KSEG, 1), 0)
    onehot = (m == iota_col).astype(jnp.float32)

    @pl.when(t == 0)
    def _():
        var_s[...] = jnp.zeros_like(var_s)

    centers = ctab_ref[0, 0:32, :]
    csq = ctab_ref[0, 32:33, :]
    dots = lax.dot_general(centers, emb, (((0,), (0,)), ((), ())),
                           preferred_element_type=jnp.float32)
    normsq = jnp.sum(emb * emb, axis=0)
    seldot = jnp.sum(onehot * dots, axis=0)
    selcsq = lax.dot_general(csq, onehot, (((1,), (0,)), ((), ())),
                             preferred_element_type=jnp.float32)[0]
    sq = jnp.maximum(normsq - 2.0 * seldot + selcsq, 0.0)
    d = jnp.sqrt(sq + EPS)
    h = jnp.maximum(d - DELTA_VAR, 0.0)
    var_s[...] += lax.dot_general((h * h)[None, :], onehot,
                                  (((1,), (1,)), ((), ())),
                                  preferred_element_type=jnp.float32)

    @pl.when(t == nT - 1)
    def _():
        out_ref[0, 0:1, :] = var_s[...]


def _finish_body(ctab_ref, scv_ref, tcv_ref, out_ref):
    kk_row = lax.broadcasted_iota(jnp.int32, (1, KSEG), 1)
    kk_sq_r = lax.broadcasted_iota(jnp.int32, (KSEG, KSEG), 1)
    kk_sq_c = lax.broadcasted_iota(jnp.int32, (KSEG, KSEG), 0)
    eye = (kk_sq_c == kk_sq_r).astype(jnp.float32)
    lv_acc = jnp.float32(0.0)
    ld_acc = jnp.float32(0.0)
    lr_acc = jnp.float32(0.0)
    vb_acc = jnp.float32(0.0)
    for bi in range(4):
        centers = ctab_ref[bi, 0:32, :]
        csq_row = ctab_ref[bi, 32:33, :]
        counts = ctab_ref[bi, 40:41, :]
        lo = 16 * (bi // 2) + (bi % 2) * 8
        varsum = tcv_ref[bi, 0:1, :] + jnp.sum(scv_ref[lo:lo + 8, :], axis=0,
                                               keepdims=True)
        valid_row = jnp.logical_and(counts > 0, kk_row > 0)
        vrf = valid_row.astype(jnp.float32)
        n_inst = jnp.sum(vrf)
        safe = jnp.where(counts > 0, counts, 1.0)
        var_per = varsum / safe
        lv = jnp.sum(jnp.where(valid_row, var_per, 0.0)) / jnp.maximum(n_inst, 1.0)
        gram = lax.dot_general(centers, centers, (((0,), (0,)), ((), ())),
                               preferred_element_type=jnp.float32)
        csq_col = jnp.sum(eye * gram, axis=1, keepdims=True)
        sq_pair = jnp.maximum(csq_col + csq_row - 2.0 * gram, 0.0)
        outer = lax.dot_general(vrf, vrf, (((0,), (0,)), ((), ())),
                                preferred_element_type=jnp.float32)
        pm = jnp.logical_and(outer > 0.5, kk_sq_c < kk_sq_r)
        pair_d = jnp.sqrt(jnp.where(pm, sq_pair, 1.0))
        hd = jnp.maximum(2.0 * DELTA_DIST - pair_d, 0.0) ** 2
        n_pairs = jnp.sum(pm.astype(jnp.float32))
        ld = jnp.sum(jnp.where(pm, hd, 0.0)) / jnp.maximum(n_pairs, 1.0)
        c_norm = jnp.sqrt(jnp.where(valid_row, csq_row, 1.0))
        lr = jnp.sum(jnp.where(valid_row, c_norm, 0.0)) / jnp.maximum(n_inst, 1.0)
        validb = (n_inst > 0).astype(jnp.float32)
        lv_acc += lv * validb
        ld_acc += ld * validb
        lr_acc += lr * validb
        vb_acc += validb
    denom = jnp.maximum(vb_acc, 1.0)
    lvt = lv_acc / denom
    ldt = ld_acc / denom
    lrt = lr_acc / denom
    total = ALPHA * lvt + BETA * ldt + GAMMA * lrt
    row = lax.broadcasted_iota(jnp.int32, (8, 128), 0)
    col = lax.broadcasted_iota(jnp.int32, (8, 128), 1)
    vals = jnp.where(col == 0, total,
           jnp.where(col == 1, lvt,
           jnp.where(col == 2, ldt, lrt)))
    out_ref[...] = jnp.where(row == 0, vals, 0.0)


def kernel(embedding, instance_mask):
    if instance_mask.ndim == 4:
        instance_mask = instance_mask[:, 0]
    B, E, H, W = embedding.shape
    P = H * W
    emb3 = embedding.reshape(B, E, P)
    mask2 = instance_mask.reshape(B, P)
    mask3 = instance_mask.reshape(B, 1, P)

    mesh = plsc.VectorSubcoreMesh(core_axis_name="c", subcore_axis_name="s")
    sc_params = pltpu.CompilerParams(needs_layout_passes=False)

    sc_p1 = functools.partial(
        pl.kernel, mesh=mesh, compiler_params=sc_params,
        out_type=jax.ShapeDtypeStruct((32, 560), jnp.float32),
        scratch_types=[
            pltpu.VMEM((NE, C), jnp.float32),
            pltpu.VMEM((NE, C), jnp.float32),
            pltpu.VMEM((C,), jnp.int32),
            pltpu.VMEM((C,), jnp.int32),
            pltpu.VMEM((16 * 528,), jnp.float32),
            pltpu.VMEM((560,), jnp.float32),
            pltpu.SemaphoreType.DMA,
            pltpu.SemaphoreType.DMA,
        ],
    )(_sc_p1_body)

    sc_p2 = functools.partial(
        pl.kernel, mesh=mesh, compiler_params=sc_params,
        out_type=jax.ShapeDtypeStruct((32, 16), jnp.float32),
        scratch_types=[
            pltpu.VMEM((NE, C), jnp.float32),
            pltpu.VMEM((NE, C), jnp.float32),
            pltpu.VMEM((C,), jnp.int32),
            pltpu.VMEM((C,), jnp.int32),
            pltpu.VMEM((640,), jnp.float32),
            pltpu.VMEM((16 * 16,), jnp.float32),
            pltpu.VMEM((16,), jnp.float32),
            pltpu.SemaphoreType.DMA,
            pltpu.SemaphoreType.DMA,
        ],
    )(_sc_p2_body)

    tc_arb = pltpu.CompilerParams(
        dimension_semantics=("arbitrary", "arbitrary"))

    scp1_out = sc_p1(emb3, mask2)
    tcp1_out = pl.pallas_call(
        _tc_p1_body,
        grid=(B, NT_TC),
        in_specs=[
            pl.BlockSpec((1, E, TILE), lambda b, t: (b, 0, t)),
            pl.BlockSpec((1, 1, TILE), lambda b, t: (b, 0, t)),
        ],
        out_specs=pl.BlockSpec((1, 48, KSEG), lambda b, t: (b, 0, 0)),
        out_shape=jax.ShapeDtypeStruct((B, 48, KSEG), jnp.float32),
        scratch_shapes=[
            pltpu.VMEM((NE, KSEG), jnp.float32),
            pltpu.VMEM((1, KSEG), jnp.float32),
        ],
        compiler_params=tc_arb,
    )(emb3, mask3)

    ctab_all = pl.pallas_call(
        _combine_body,
        grid=(1,),
        in_specs=[
            pl.BlockSpec((B, 48, KSEG), lambda i: (0, 0, 0)),
            pl.BlockSpec((32, 35, KSEG), lambda i: (0, 0, 0)),
        ],
        out_specs=pl.BlockSpec((B, 48, KSEG), lambda i: (0, 0, 0)),
        out_shape=jax.ShapeDtypeStruct((B, 48, KSEG), jnp.float32),
    )(tcp1_out, scp1_out.reshape(32, 35, KSEG))

    ctab_flat = ctab_all.reshape(B * 48 * KSEG)

    scv_out = sc_p2(emb3, mask2, ctab_flat)
    tcv_out = pl.pallas_call(
        _tc_p2_body,
        grid=(B, NT_TC),
        in_specs=[
            pl.BlockSpec((1, E, TILE), lambda b, t: (b, 0, t)),
            pl.BlockSpec((1, 1, TILE), lambda b, t: (b, 0, t)),
            pl.BlockSpec((1, 48, KSEG), lambda b, t: (b, 0, 0)),
        ],
        out_specs=pl.BlockSpec((1, 1, KSEG), lambda b, t: (b, 0, 0)),
        out_shape=jax.ShapeDtypeStruct((B, 1, KSEG), jnp.float32),
        scratch_shapes=[
            pltpu.VMEM((1, KSEG), jnp.float32),
        ],
        compiler_params=tc_arb,
    )(emb3, mask3, ctab_all)

    out = pl.pallas_call(
        _finish_body,
        grid=(1,),
        in_specs=[
            pl.BlockSpec((B, 48, KSEG), lambda i: (0, 0, 0)),
            pl.BlockSpec((32, KSEG), lambda i: (0, 0)),
            pl.BlockSpec((B, 1, KSEG), lambda i: (0, 0, 0)),
        ],
        out_specs=pl.BlockSpec((8, 128), lambda i: (0, 0)),
        out_shape=jax.ShapeDtypeStruct((8, 128), jnp.float32),
    )(ctab_all, scv_out, tcv_out)
    return (out[0, 0], out[0, 1], out[0, 2], out[0, 3])
